# baseline (speedup 1.0000x reference)
import functools

import jax
import jax.numpy as jnp
from jax import lax
from jax.experimental import pallas as pl
from jax.experimental.pallas import tpu as pltpu
from jax.experimental.pallas import tpu_sc as plsc

NC = 2
NS = 16
NW = NC * NS
LANES = 16
CHUNK = 128


def _sc_pool(x_chunks, emb_table, B, L, D, CB, n_chunks):
    bags_per_w = B // NW
    dregs = D // LANES
    inv_l = 1.0 / L
    mesh = plsc.VectorSubcoreMesh(
        core_axis_name="c", subcore_axis_name="s", num_cores=NC, num_subcores=NS
    )

    @functools.partial(
        pl.kernel,
        out_type=jax.ShapeDtypeStruct((B, D), jnp.float32),
        mesh=mesh,
        compiler_params=pltpu.CompilerParams(use_tc_tiling_on_sc=False),
        scratch_types=[
            pltpu.VMEM((n_chunks, CHUNK), jnp.int32),
            pltpu.VMEM((CHUNK, 2 * D), jnp.float32),
            pltpu.VMEM((CHUNK, 2 * D), jnp.float32),
            pltpu.VMEM((bags_per_w, D), jnp.float32),
            pltpu.SemaphoreType.DMA,
            pltpu.SemaphoreType.DMA,
        ],
    )
    def k(idx_hbm, table_hbm, out_hbm, idx_v, buf0, buf1, pooled_v, sem0, sem1):
        wid = lax.axis_index("s") * NC + lax.axis_index("c")
        pltpu.sync_copy(idx_hbm.at[wid], idx_v)

        def accumulate(buf, g):
            def l_body(l, accs):
                out = []
                for bag in range(CB):
                    for dd in range(dregs):
                        v = buf[bag * L + l, pl.ds(dd * LANES, LANES)]
                        out.append(accs[bag * dregs + dd] + v)
                return tuple(out)

            zero = tuple(
                jnp.zeros((LANES,), jnp.float32) for _ in range(CB * dregs)
            )
            accs = lax.fori_loop(0, L, l_body, zero)
            for bag in range(CB):
                for dd in range(dregs):
                    pooled_v[g * CB + bag, pl.ds(dd * LANES, LANES)] = (
                        accs[bag * dregs + dd] * inv_l
                    )

        pltpu.async_copy(table_hbm.at[idx_v.at[0]], buf0, sem0)

        def pair_body(p, carry):
            g0 = 2 * p
            g1 = g0 + 1
            pltpu.async_copy(table_hbm.at[idx_v.at[g1]], buf1, sem1)
            pltpu.make_async_copy(table_hbm.at[idx_v.at[g0]], buf0, sem0).wait()
            accumulate(buf0, g0)
            g2 = (g0 + 2) % n_chunks
            pltpu.async_copy(table_hbm.at[idx_v.at[g2]], buf0, sem0)
            pltpu.make_async_copy(table_hbm.at[idx_v.at[g1]], buf1, sem1).wait()
            accumulate(buf1, g1)
            return carry

        lax.fori_loop(0, n_chunks // 2, pair_body, 0)
        pltpu.make_async_copy(table_hbm.at[idx_v.at[0]], buf0, sem0).wait()
        pltpu.sync_copy(pooled_v, out_hbm.at[pl.ds(wid * bags_per_w, bags_per_w)])

    return k(x_chunks, emb_table)


def _tc_head(pooled, wt, b2, B, D, OUT):
    BB = 1024

    def body(p_ref, w_ref, b_ref, o_ref):
        y = jnp.dot(p_ref[...], w_ref[...], preferred_element_type=jnp.float32)
        y = y + b_ref[...]
        m = jnp.max(y, axis=1, keepdims=True)
        e = jnp.exp(y - m)
        o_ref[...] = e / jnp.sum(e, axis=1, keepdims=True)

    return pl.pallas_call(
        body,
        grid=(B // BB,),
        in_specs=[
            pl.BlockSpec((BB, D), lambda i: (i, 0)),
            pl.BlockSpec((D, OUT), lambda i: (0, 0)),
            pl.BlockSpec((1, OUT), lambda i: (0, 0)),
        ],
        out_specs=pl.BlockSpec((BB, OUT), lambda i: (i, 0)),
        out_shape=jax.ShapeDtypeStruct((B, OUT), jnp.float32),
    )(pooled, wt, b2)


def kernel(x, emb_table, W, b):
    B, L = x.shape
    _, D = emb_table.shape
    OUT = W.shape[0]
    CB = 2
    n_chunks = B // (NW * CB)
    xr = x.astype(jnp.int32).reshape(B // CB, CB * L)
    xp = jnp.concatenate([xr, xr[:, : CHUNK - CB * L]], axis=1)
    x_chunks = xp.reshape(NW, n_chunks, CHUNK)
    tab = jnp.pad(emb_table, ((0, 0), (0, 2 * D - D)))
    pooled = _sc_pool(x_chunks, tab, B, L, D, CB, n_chunks)
    return _tc_head(pooled, W.T, b.reshape(1, OUT), B, D, OUT)


# speedup vs baseline: 1.0071x; 1.0071x over previous
import functools

import jax
import jax.numpy as jnp
from jax import lax
from jax.experimental import pallas as pl
from jax.experimental.pallas import tpu as pltpu
from jax.experimental.pallas import tpu_sc as plsc

NC = 2
NS = 16
NW = NC * NS
LANES = 16
CHUNK = 128


def _sc_pool(x_chunks, emb_table, B, L, D, CB, n_chunks):
    bags_per_w = B // NW
    dregs = D // LANES
    inv_l = 1.0 / L
    mesh = plsc.VectorSubcoreMesh(
        core_axis_name="c", subcore_axis_name="s", num_cores=NC, num_subcores=NS
    )

    @functools.partial(
        pl.kernel,
        out_type=jax.ShapeDtypeStruct((B, D), jnp.float32),
        mesh=mesh,
        scratch_types=[
            pltpu.VMEM((n_chunks, CHUNK), jnp.int32),
            pltpu.VMEM((CHUNK, 2 * D), jnp.float32),
            pltpu.VMEM((CHUNK, 2 * D), jnp.float32),
            pltpu.VMEM((bags_per_w, D), jnp.float32),
            pltpu.SemaphoreType.DMA,
            pltpu.SemaphoreType.DMA,
        ],
    )
    def k(idx_hbm, table_hbm, out_hbm, idx_v, buf0, buf1, pooled_v, sem0, sem1):
        wid = lax.axis_index("s") * NC + lax.axis_index("c")
        pltpu.sync_copy(idx_hbm.at[wid], idx_v)

        def accumulate(buf, g):
            def l_body(l, accs):
                out = []
                for bag in range(CB):
                    for dd in range(dregs):
                        v = buf[bag * L + l, pl.ds(dd * LANES, LANES)]
                        out.append(accs[bag * dregs + dd] + v)
                return tuple(out)

            zero = tuple(
                jnp.zeros((LANES,), jnp.float32) for _ in range(CB * dregs)
            )
            accs = lax.fori_loop(0, L, l_body, zero)
            for bag in range(CB):
                for dd in range(dregs):
                    pooled_v[g * CB + bag, pl.ds(dd * LANES, LANES)] = (
                        accs[bag * dregs + dd] * inv_l
                    )

        pltpu.async_copy(table_hbm.at[idx_v.at[0]], buf0, sem0)

        def pair_body(p, carry):
            g0 = 2 * p
            g1 = g0 + 1
            pltpu.async_copy(table_hbm.at[idx_v.at[g1]], buf1, sem1)
            pltpu.make_async_copy(table_hbm.at[idx_v.at[g0]], buf0, sem0).wait()
            accumulate(buf0, g0)
            g2 = (g0 + 2) % n_chunks
            pltpu.async_copy(table_hbm.at[idx_v.at[g2]], buf0, sem0)
            pltpu.make_async_copy(table_hbm.at[idx_v.at[g1]], buf1, sem1).wait()
            accumulate(buf1, g1)
            return carry

        lax.fori_loop(0, n_chunks // 2, pair_body, 0)
        pltpu.make_async_copy(table_hbm.at[idx_v.at[0]], buf0, sem0).wait()
        pltpu.sync_copy(pooled_v, out_hbm.at[pl.ds(wid * bags_per_w, bags_per_w)])

    return k(x_chunks, emb_table)


def _tc_head(pooled, wt, b2, B, D, OUT):
    BB = 1024

    def body(p_ref, w_ref, b_ref, o_ref):
        y = jnp.dot(p_ref[...], w_ref[...], preferred_element_type=jnp.float32)
        y = y + b_ref[...]
        m = jnp.max(y, axis=1, keepdims=True)
        e = jnp.exp(y - m)
        o_ref[...] = e / jnp.sum(e, axis=1, keepdims=True)

    return pl.pallas_call(
        body,
        grid=(B // BB,),
        in_specs=[
            pl.BlockSpec((BB, D), lambda i: (i, 0)),
            pl.BlockSpec((D, OUT), lambda i: (0, 0)),
            pl.BlockSpec((1, OUT), lambda i: (0, 0)),
        ],
        out_specs=pl.BlockSpec((BB, OUT), lambda i: (i, 0)),
        out_shape=jax.ShapeDtypeStruct((B, OUT), jnp.float32),
    )(pooled, wt, b2)


def kernel(x, emb_table, W, b):
    B, L = x.shape
    _, D = emb_table.shape
    OUT = W.shape[0]
    CB = 2
    n_chunks = B // (NW * CB)
    xr = x.astype(jnp.int32).reshape(B // CB, CB * L)
    xp = jnp.concatenate([xr, xr[:, : CHUNK - CB * L]], axis=1)
    x_chunks = xp.reshape(NW, n_chunks, CHUNK)
    tab = jnp.pad(emb_table, ((0, 0), (0, 2 * D - D)))
    pooled = _sc_pool(x_chunks, tab, B, L, D, CB, n_chunks)
    return _tc_head(pooled, W.T, b.reshape(1, OUT), B, D, OUT)


# speedup vs baseline: 1.1282x; 1.1202x over previous
import functools

import jax
import jax.numpy as jnp
from jax import lax
from jax.experimental import pallas as pl
from jax.experimental.pallas import tpu as pltpu
from jax.experimental.pallas import tpu_sc as plsc

NC = 2
NS = 16
NW = NC * NS
LANES = 16
CHUNK = 128


def _sc_pool(x_chunks, emb_table, B, L, D, CB, n_chunks):
    bags_per_w = B // NW
    dregs = D // LANES
    inv_l = 1.0 / L
    mesh = plsc.VectorSubcoreMesh(
        core_axis_name="c", subcore_axis_name="s", num_cores=NC, num_subcores=NS
    )

    @functools.partial(
        pl.kernel,
        out_type=jax.ShapeDtypeStruct((B, D), jnp.float32),
        mesh=mesh,
        compiler_params=pltpu.CompilerParams(use_tc_tiling_on_sc=False),
        scratch_types=[
            pltpu.VMEM((n_chunks, CHUNK), jnp.int32),
            pltpu.VMEM((CHUNK, D), jnp.float32),
            pltpu.VMEM((CHUNK, D), jnp.float32),
            pltpu.VMEM((CHUNK, D), jnp.float32),
            pltpu.VMEM((CHUNK, D), jnp.float32),
            pltpu.VMEM((bags_per_w, D), jnp.float32),
            pltpu.SemaphoreType.DMA,
            pltpu.SemaphoreType.DMA,
            pltpu.SemaphoreType.DMA,
            pltpu.SemaphoreType.DMA,
        ],
    )
    def k(idx_hbm, table_hbm, out_hbm, idx_v, b0, b1, b2, b3, pooled_v, s0, s1, s2, s3):
        bufs = (b0, b1, b2, b3)
        sems = (s0, s1, s2, s3)
        wid = lax.axis_index("s") * NC + lax.axis_index("c")
        pltpu.sync_copy(idx_hbm.at[wid], idx_v)

        def accumulate(buf, g):
            def l_body(l, accs):
                out = []
                for bag in range(CB):
                    for dd in range(dregs):
                        v = buf[bag * L + l, pl.ds(dd * LANES, LANES)]
                        out.append(accs[bag * dregs + dd] + v)
                return tuple(out)

            zero = tuple(
                jnp.zeros((LANES,), jnp.float32) for _ in range(CB * dregs)
            )
            accs = lax.fori_loop(0, L, l_body, zero)
            for bag in range(CB):
                for dd in range(dregs):
                    pooled_v[g * CB + bag, pl.ds(dd * LANES, LANES)] = (
                        accs[bag * dregs + dd] * inv_l
                    )

        NBUF = 4
        for b in range(NBUF - 1):
            pltpu.async_copy(table_hbm.at[idx_v.at[b]], bufs[b], sems[b])

        def round_body(p, carry):
            base = NBUF * p
            for b in range(NBUF):
                g = base + b
                gn = (g + NBUF - 1) % n_chunks
                pltpu.async_copy(table_hbm.at[idx_v.at[gn]], bufs[(b + NBUF - 1) % NBUF], sems[(b + NBUF - 1) % NBUF])
                pltpu.make_async_copy(table_hbm.at[idx_v.at[g]], bufs[b], sems[b]).wait()
                accumulate(bufs[b], g)
            return carry

        lax.fori_loop(0, n_chunks // NBUF, round_body, 0)
        for b in range(NBUF - 1):
            pltpu.make_async_copy(table_hbm.at[idx_v.at[b]], bufs[b], sems[b]).wait()
        pltpu.sync_copy(pooled_v, out_hbm.at[pl.ds(wid * bags_per_w, bags_per_w)])

    return k(x_chunks, emb_table)


def _tc_head(pooled, wt, b2, B, D, OUT):
    BB = 1024

    def body(p_ref, w_ref, b_ref, o_ref):
        y = jnp.dot(p_ref[...], w_ref[...], preferred_element_type=jnp.float32)
        y = y + b_ref[...]
        m = jnp.max(y, axis=1, keepdims=True)
        e = jnp.exp(y - m)
        o_ref[...] = e / jnp.sum(e, axis=1, keepdims=True)

    return pl.pallas_call(
        body,
        grid=(B // BB,),
        in_specs=[
            pl.BlockSpec((BB, D), lambda i: (i, 0)),
            pl.BlockSpec((D, OUT), lambda i: (0, 0)),
            pl.BlockSpec((1, OUT), lambda i: (0, 0)),
        ],
        out_specs=pl.BlockSpec((BB, OUT), lambda i: (i, 0)),
        out_shape=jax.ShapeDtypeStruct((B, OUT), jnp.float32),
    )(pooled, wt, b2)


def kernel(x, emb_table, W, b):
    B, L = x.shape
    _, D = emb_table.shape
    OUT = W.shape[0]
    CB = 2
    n_chunks = B // (NW * CB)
    xr = x.astype(jnp.int32).reshape(B // CB, CB * L)
    xp = jnp.concatenate([xr, xr[:, : CHUNK - CB * L]], axis=1)
    x_chunks = xp.reshape(NW, n_chunks, CHUNK)
    pooled = _sc_pool(x_chunks, emb_table, B, L, D, CB, n_chunks)
    return _tc_head(pooled, W.T, b.reshape(1, OUT), B, D, OUT)


# speedup vs baseline: 1.1437x; 1.0137x over previous
import functools

import jax
import jax.numpy as jnp
from jax import lax
from jax.experimental import pallas as pl
from jax.experimental.pallas import tpu as pltpu
from jax.experimental.pallas import tpu_sc as plsc

NC = 2
NS = 16
NW = NC * NS
LANES = 16
CHUNK = 128


def _sc_pool(x_chunks, emb_table, B, L, D, CB, n_chunks):
    bags_per_w = B // NW
    dregs = D // LANES
    inv_l = 1.0 / L
    mesh = plsc.VectorSubcoreMesh(
        core_axis_name="c", subcore_axis_name="s", num_cores=NC, num_subcores=NS
    )

    @functools.partial(
        pl.kernel,
        out_type=jax.ShapeDtypeStruct((B, D), jnp.float32),
        mesh=mesh,
        compiler_params=pltpu.CompilerParams(use_tc_tiling_on_sc=False),
        scratch_types=[
            pltpu.VMEM((n_chunks, CHUNK), jnp.int32),
            pltpu.VMEM((CHUNK, D), jnp.float32),
            pltpu.VMEM((CHUNK, D), jnp.float32),
            pltpu.VMEM((CHUNK, D), jnp.float32),
            pltpu.VMEM((CHUNK, D), jnp.float32),
            pltpu.VMEM((CHUNK, D), jnp.float32),
            pltpu.VMEM((CHUNK, D), jnp.float32),
            pltpu.VMEM((bags_per_w, D), jnp.float32),
            pltpu.SemaphoreType.DMA,
            pltpu.SemaphoreType.DMA,
            pltpu.SemaphoreType.DMA,
            pltpu.SemaphoreType.DMA,
            pltpu.SemaphoreType.DMA,
            pltpu.SemaphoreType.DMA,
        ],
    )
    def k(idx_hbm, table_hbm, out_hbm, idx_v, b0, b1, b2, b3, b4, b5,
          pooled_v, s0, s1, s2, s3, s4, s5):
        bufs = (b0, b1, b2, b3, b4, b5)
        sems = (s0, s1, s2, s3, s4, s5)
        wid = lax.axis_index("s") * NC + lax.axis_index("c")
        pltpu.sync_copy(idx_hbm.at[wid], idx_v)

        def accumulate(buf, g):
            def l_body(l, accs):
                out = []
                for bag in range(CB):
                    for dd in range(dregs):
                        v = buf[bag * L + l, pl.ds(dd * LANES, LANES)]
                        out.append(accs[bag * dregs + dd] + v)
                return tuple(out)

            zero = tuple(
                jnp.zeros((LANES,), jnp.float32) for _ in range(CB * dregs)
            )
            accs = lax.fori_loop(0, L, l_body, zero)
            for bag in range(CB):
                for dd in range(dregs):
                    pooled_v[g * CB + bag, pl.ds(dd * LANES, LANES)] = (
                        accs[bag * dregs + dd] * inv_l
                    )

        NBUF = 6
        R = n_chunks // NBUF
        REM = n_chunks - R * NBUF
        for b in range(NBUF - 1):
            pltpu.async_copy(table_hbm.at[idx_v.at[b]], bufs[b], sems[b])

        def round_body(p, carry):
            base = NBUF * p
            for b in range(NBUF):
                g = base + b
                gn = (g + NBUF - 1) % n_chunks
                bn = (b + NBUF - 1) % NBUF
                pltpu.async_copy(table_hbm.at[idx_v.at[gn]], bufs[bn], sems[bn])
                pltpu.make_async_copy(table_hbm.at[idx_v.at[g]], bufs[b], sems[b]).wait()
                accumulate(bufs[b], g)
            return carry

        lax.fori_loop(0, R, round_body, 0)
        for j in range(REM):
            g = R * NBUF + j
            bj = g % NBUF
            pltpu.make_async_copy(table_hbm.at[idx_v.at[g]], bufs[bj], sems[bj]).wait()
            accumulate(bufs[bj], g)
        for j in range(NBUF - 1 - REM):
            bj = (REM + j) % NBUF
            pltpu.make_async_copy(table_hbm.at[idx_v.at[j]], bufs[bj], sems[bj]).wait()
        pltpu.sync_copy(pooled_v, out_hbm.at[pl.ds(wid * bags_per_w, bags_per_w)])

    return k(x_chunks, emb_table)


def _tc_head(pooled, wt, b2, B, D, OUT):
    BB = 1024

    def body(p_ref, w_ref, b_ref, o_ref):
        y = jnp.dot(p_ref[...], w_ref[...], preferred_element_type=jnp.float32)
        y = y + b_ref[...]
        m = jnp.max(y, axis=1, keepdims=True)
        e = jnp.exp(y - m)
        o_ref[...] = e / jnp.sum(e, axis=1, keepdims=True)

    return pl.pallas_call(
        body,
        grid=(B // BB,),
        in_specs=[
            pl.BlockSpec((BB, D), lambda i: (i, 0)),
            pl.BlockSpec((D, OUT), lambda i: (0, 0)),
            pl.BlockSpec((1, OUT), lambda i: (0, 0)),
        ],
        out_specs=pl.BlockSpec((BB, OUT), lambda i: (i, 0)),
        out_shape=jax.ShapeDtypeStruct((B, OUT), jnp.float32),
    )(pooled, wt, b2)


def kernel(x, emb_table, W, b):
    B, L = x.shape
    _, D = emb_table.shape
    OUT = W.shape[0]
    CB = 2
    n_chunks = B // (NW * CB)
    xr = x.astype(jnp.int32).reshape(B // CB, CB * L)
    xp = jnp.concatenate([xr, xr[:, : CHUNK - CB * L]], axis=1)
    x_chunks = xp.reshape(NW, n_chunks, CHUNK)
    pooled = _sc_pool(x_chunks, emb_table, B, L, D, CB, n_chunks)
    return _tc_head(pooled, W.T, b.reshape(1, OUT), B, D, OUT)
